# Initial kernel scaffold; baseline (speedup 1.0000x reference)
import jax
import jax.numpy as jnp
from jax.experimental import pallas as pl


def kernel(queue, keys, ptr):
    raise NotImplementedError("write your pallas kernel here")


# speedup vs baseline: 3.6015x; 3.6015x over previous
import jax
import jax.numpy as jnp
from jax import lax
from jax.experimental import pallas as pl
from jax.experimental.pallas import tpu as pltpu
from jax.experimental.pallas import tpu_sc as plsc

_Q = 100000
_H = 768
_B = 4096
_NC = 2
_NS = 16
_NW = _NC * _NS
_RPW = _B // _NW
_L = 16


def _enqueue_body(ptr_hbm, keys_hbm, queue_ref, ptr_v, idx_v, rows_v, sem):
    wid = lax.axis_index("s") * _NC + lax.axis_index("c")
    base = wid * _RPW
    pltpu.sync_copy(ptr_hbm, ptr_v)
    ptr_vec = ptr_v[...]
    iota = lax.iota(jnp.int32, _L)
    for j in range(_RPW // _L):
        idx_v[pl.ds(j * _L, _L)] = lax.rem(ptr_vec + (base + j * _L) + iota, _Q)
    pltpu.sync_copy(keys_hbm.at[pl.ds(base, _RPW)], rows_v)
    pltpu.async_copy(rows_v, queue_ref.at[idx_v], sem).wait()


def kernel(queue, keys, ptr):
    ptr32 = jnp.asarray(ptr, jnp.int32)
    ptr_arr = jnp.full((_L,), ptr32, jnp.int32)
    mesh = plsc.VectorSubcoreMesh(
        core_axis_name="c", subcore_axis_name="s", num_cores=_NC
    )
    enqueue = pl.kernel(
        _enqueue_body,
        out_type=(),
        mesh=mesh,
        scratch_types=[
            pltpu.VMEM((_L,), jnp.int32),
            pltpu.VMEM((_RPW,), jnp.int32),
            pltpu.VMEM((_RPW, _H), jnp.float32),
            pltpu.SemaphoreType.DMA,
        ],
    )
    qref = jax.new_ref(queue)
    enqueue(ptr_arr, keys, qref)
    new_queue = qref[...]
    new_ptr = lax.rem(ptr32 + _B, _Q)
    return new_queue, new_ptr
